# baseline (speedup 1.0000x reference)
import dataclasses
import functools

import jax
import jax.numpy as jnp
from jax import lax
from jax.experimental import pallas as pl
from jax.experimental.pallas import tpu as pltpu
from jax.experimental.pallas import tpu_sc as plsc

NUM_CORES = 2
NUM_SUBCORES = 16
NUM_WORKERS = NUM_CORES * NUM_SUBCORES
BATCH = 16384
DIM = 16
B_PER_W = BATCH // NUM_WORKERS


def kernel(train_x, user_weight, item_weight):
    uid = train_x[:, 0]
    iid = train_x[:, 1]
    mesh = plsc.VectorSubcoreMesh(core_axis_name="c", subcore_axis_name="s")
    cparams = pltpu.CompilerParams(
        needs_layout_passes=False, use_tc_tiling_on_sc=False
    )

    @functools.partial(
        pl.kernel,
        out_type=jax.ShapeDtypeStruct((BATCH,), jnp.float32),
        mesh=mesh,
        compiler_params=cparams,
        scratch_types=[
            pltpu.VMEM((B_PER_W,), jnp.int32),
            pltpu.VMEM((B_PER_W,), jnp.int32),
            pltpu.VMEM((B_PER_W, DIM), jnp.float32),
            pltpu.VMEM((B_PER_W, DIM), jnp.float32),
            pltpu.VMEM((B_PER_W,), jnp.float32),
            pltpu.SemaphoreType.DMA,
            pltpu.SemaphoreType.DMA,
        ],
    )
    def sc_kernel(uid_hbm, iid_hbm, uw_hbm, iw_hbm, out_hbm,
                  uidx_v, iidx_v, u_v, i_v, o_v, sem_u, sem_i):
        wid = lax.axis_index("s") * NUM_CORES + lax.axis_index("c")
        base = wid * B_PER_W
        pltpu.sync_copy(uid_hbm.at[pl.ds(base, B_PER_W)], uidx_v)
        pltpu.sync_copy(iid_hbm.at[pl.ds(base, B_PER_W)], iidx_v)
        cp_u = pltpu.async_copy(uw_hbm.at[uidx_v], u_v, sem_u)
        cp_i = pltpu.async_copy(iw_hbm.at[iidx_v], i_v, sem_i)
        cp_u.wait()
        cp_i.wait()

        lane = lax.iota(jnp.int32, 16)
        mask_last = lane == 15

        @pl.loop(0, B_PER_W)
        def _(w):
            p = u_v[w, :] * i_v[w, :]
            s = jnp.cumsum(p)
            idx = jnp.full((16,), w, jnp.int32)
            plsc.store_scatter(o_v, [idx], s, mask=mask_last)

        pltpu.sync_copy(o_v, out_hbm.at[pl.ds(base, B_PER_W)])

    return sc_kernel(uid, iid, user_weight, item_weight)


# speedup vs baseline: 6.3730x; 6.3730x over previous
import functools

import jax
import jax.numpy as jnp
from jax import lax
from jax.experimental import pallas as pl
from jax.experimental.pallas import tpu as pltpu
from jax.experimental.pallas import tpu_sc as plsc

NUM_CORES = 2
NUM_SUBCORES = 16
NUM_WORKERS = NUM_CORES * NUM_SUBCORES
BATCH = 16384
DIM = 16
B_PER_W = BATCH // NUM_WORKERS
GROUP = 8
NPHASE = 3
NSLOT = NPHASE * 2 * GROUP


def kernel(train_x, user_weight, item_weight):
    uid = train_x[:, 0]
    iid = train_x[:, 1]
    ut = user_weight.T
    it = item_weight.T
    mesh = plsc.VectorSubcoreMesh(core_axis_name="c", subcore_axis_name="s")
    cparams = pltpu.CompilerParams(needs_layout_passes=False)

    @functools.partial(
        pl.kernel,
        out_type=jax.ShapeDtypeStruct((BATCH,), jnp.float32),
        mesh=mesh,
        compiler_params=cparams,
        scratch_types=[
            pltpu.VMEM((B_PER_W,), jnp.int32),
            pltpu.VMEM((B_PER_W,), jnp.int32),
            pltpu.VMEM((NSLOT, DIM, 128), jnp.float32),
            pltpu.VMEM((B_PER_W,), jnp.float32),
            pltpu.SemaphoreType.DMA,
            pltpu.SemaphoreType.DMA,
            pltpu.SemaphoreType.DMA,
        ],
    )
    def sc_kernel(uid_hbm, iid_hbm, ut_hbm, it_hbm, out_hbm,
                  uidx_v, iidx_v, blk_v, o_v, sem0, sem1, sem2):
        sems = (sem0, sem1, sem2)
        wid = lax.axis_index("s") * NUM_CORES + lax.axis_index("c")
        base = wid * B_PER_W
        pltpu.sync_copy(uid_hbm.at[pl.ds(base, B_PER_W)], uidx_v)
        pltpu.sync_copy(iid_hbm.at[pl.ds(base, B_PER_W)], iidx_v)

        lane = lax.iota(jnp.int32, 16)
        mask_last = lane == 15

        def fire(phase, uvec, ivec, joff):
            for j in range(GROUP):
                slot = phase * 2 * GROUP + 2 * j
                ru = uvec[joff + j]
                ri = ivec[joff + j]
                cu = pl.multiple_of(ru & ~127, 128)
                ci = pl.multiple_of(ri & ~127, 128)
                pltpu.async_copy(
                    ut_hbm.at[:, pl.ds(cu, 128)], blk_v.at[slot],
                    sems[phase])
                pltpu.async_copy(
                    it_hbm.at[:, pl.ds(ci, 128)], blk_v.at[slot + 1],
                    sems[phase])

        def drain(phase):
            for j in range(2 * GROUP):
                slot = phase * 2 * GROUP + j
                pltpu.make_async_copy(
                    ut_hbm.at[:, pl.ds(0, 128)], blk_v.at[slot], sems[phase]
                ).wait()

        def extract(phase, uvec, ivec, joff, w0):
            for j in range(GROUP):
                slot = phase * 2 * GROUP + 2 * j
                ru = uvec[joff + j]
                ri = ivec[joff + j]
                ucol = jnp.full((16,), ru & 127, jnp.int32)
                icol = jnp.full((16,), ri & 127, jnp.int32)
                uslot = jnp.full((16,), slot, jnp.int32)
                islot = jnp.full((16,), slot + 1, jnp.int32)
                urow = plsc.load_gather(blk_v, [uslot, lane, ucol])
                irow = plsc.load_gather(blk_v, [islot, lane, icol])
                s = jnp.cumsum(urow * irow)
                widx = jnp.full((16,), w0 + j, jnp.int32)
                plsc.store_scatter(o_v, [widx], s, mask=mask_last)

        pvec_u = uidx_v[pl.ds(0, 16)]
        pvec_i = iidx_v[pl.ds(0, 16)]
        qvec_u = uidx_v[pl.ds(16, 16)]
        qvec_i = iidx_v[pl.ds(16, 16)]
        fire(0, pvec_u, pvec_i, 0)
        fire(1, pvec_u, pvec_i, GROUP)
        fire(2, qvec_u, qvec_i, 0)

        @pl.loop(0, 480, step=24)
        def _(w0):
            va_u = uidx_v[pl.ds(w0, 16)]
            va_i = iidx_v[pl.ds(w0, 16)]
            vb_u = uidx_v[pl.ds(w0 + 16, 16)]
            vb_i = iidx_v[pl.ds(w0 + 16, 16)]
            vf1_u = uidx_v[pl.ds(w0 + 24, 16)]
            vf1_i = iidx_v[pl.ds(w0 + 24, 16)]
            vf2_u = uidx_v[pl.ds(w0 + 40, 16)]
            vf2_i = iidx_v[pl.ds(w0 + 40, 16)]
            drain(0)
            extract(0, va_u, va_i, 0, w0)
            fire(0, vf1_u, vf1_i, 0)
            drain(1)
            extract(1, va_u, va_i, GROUP, w0 + GROUP)
            fire(1, vf1_u, vf1_i, GROUP)
            drain(2)
            extract(2, vb_u, vb_i, 0, w0 + 16)
            fire(2, vf2_u, vf2_i, 0)

        t1_u = uidx_v[pl.ds(480, 16)]
        t1_i = iidx_v[pl.ds(480, 16)]
        t2_u = uidx_v[pl.ds(496, 16)]
        t2_i = iidx_v[pl.ds(496, 16)]
        drain(0)
        extract(0, t1_u, t1_i, 0, 480)
        fire(0, t2_u, t2_i, GROUP)
        drain(1)
        extract(1, t1_u, t1_i, GROUP, 488)
        drain(2)
        extract(2, t2_u, t2_i, 0, 496)
        drain(0)
        extract(0, t2_u, t2_i, GROUP, 504)

        pltpu.sync_copy(o_v, out_hbm.at[pl.ds(base, B_PER_W)])

    return sc_kernel(uid, iid, ut, it)
